# baseline (speedup 1.0000x reference)
import jax
import jax.numpy as jnp
from jax.experimental import pallas as pl
from jax.experimental.pallas import tpu as pltpu

HID = 4096
NE = 64
BT = 1024


def _router_body(x_ref, w_ref, b_ref, o_ref):
    x = x_ref[...]
    w = w_ref[...]
    logits = jax.lax.dot_general(
        x, w, (((1,), (0,)), ((), ())), preferred_element_type=jnp.float32
    )
    logits = logits + b_ref[...]
    m = jnp.max(logits, axis=-1, keepdims=True)
    e = jnp.exp(logits - m)
    o_ref[...] = e / jnp.sum(e, axis=-1, keepdims=True)


def kernel(x, W, b):
    tokens = x.shape[0]
    return pl.pallas_call(
        _router_body,
        grid=(tokens // BT,),
        in_specs=[
            pl.BlockSpec((BT, HID), lambda i: (i, 0)),
            pl.BlockSpec((HID, NE), lambda i: (0, 0)),
            pl.BlockSpec((1, NE), lambda i: (0, 0)),
        ],
        out_specs=pl.BlockSpec((BT, NE), lambda i: (i, 0)),
        out_shape=jax.ShapeDtypeStruct((tokens, NE), jnp.float32),
        compiler_params=pltpu.CompilerParams(
            dimension_semantics=("parallel",),
        ),
    )(x, W.T, b.reshape(1, NE))


# speedup vs baseline: 1.0093x; 1.0093x over previous
import jax
import jax.numpy as jnp
from jax.experimental import pallas as pl
from jax.experimental.pallas import tpu as pltpu

HID = 4096
NE = 64
BT = 1024


def _router_body(x_ref, w_ref, b_ref, o_ref):
    x = x_ref[...]
    w = w_ref[...]
    logits = jax.lax.dot_general(
        x, w, (((1,), (1,)), ((), ())), preferred_element_type=jnp.float32
    )
    logits = logits + b_ref[...]
    m = jnp.max(logits, axis=-1, keepdims=True)
    e = jnp.exp(logits - m)
    o_ref[...] = e / jnp.sum(e, axis=-1, keepdims=True)


def kernel(x, W, b):
    tokens = x.shape[0]
    return pl.pallas_call(
        _router_body,
        grid=(tokens // BT,),
        in_specs=[
            pl.BlockSpec((BT, HID), lambda i: (i, 0)),
            pl.BlockSpec((NE, HID), lambda i: (0, 0)),
            pl.BlockSpec((1, NE), lambda i: (0, 0)),
        ],
        out_specs=pl.BlockSpec((BT, NE), lambda i: (i, 0)),
        out_shape=jax.ShapeDtypeStruct((tokens, NE), jnp.float32),
        compiler_params=pltpu.CompilerParams(
            dimension_semantics=("parallel",),
        ),
    )(x, W, b.reshape(1, NE))
